# baseline (speedup 1.0000x reference)
import jax
import jax.numpy as jnp
from jax.experimental import pallas as pl
from jax.experimental.pallas import tpu as pltpu


def _make_kernel(n, d_out, bs, nbuf):
    nblk = n // bs

    def _fused_kernel(x_ref, w_ref, adj_ref, b_ref, out_ref,
                      xw_ref, bufs_ref, sems):
        def _copy(i, slot):
            return pltpu.make_async_copy(
                adj_ref.at[pl.ds(i * bs, bs), :],
                bufs_ref.at[slot],
                sems.at[slot])

        for w in range(min(nbuf, nblk)):
            _copy(w, w).start()

        xw_ref[...] = jnp.dot(x_ref[...], w_ref[...],
                              preferred_element_type=jnp.float32)

        def body(i, carry):
            slot = jax.lax.rem(i, nbuf)
            _copy(i, slot).wait()
            acc = jnp.dot(bufs_ref[slot], xw_ref[...],
                          preferred_element_type=jnp.float32)
            out_ref[pl.ds(i * bs, bs), :] = jnp.maximum(
                acc + b_ref[...], 0.0)

            @pl.when(i + nbuf < nblk)
            def _():
                _copy(i + nbuf, slot).start()

            return carry

        jax.lax.fori_loop(0, nblk, body, 0)

    return _fused_kernel


def kernel(input, adj, W, b):
    x = input.reshape(input.shape[-2], input.shape[-1])
    n, d_in = x.shape
    d_out = W.shape[1]

    bs = min(200, n)
    nblk = n // bs
    nbuf = min(4, nblk)

    out = pl.pallas_call(
        _make_kernel(n, d_out, bs, nbuf),
        in_specs=[
            pl.BlockSpec((n, d_in), lambda: (0, 0)),
            pl.BlockSpec((d_in, d_out), lambda: (0, 0)),
            pl.BlockSpec(memory_space=pl.ANY),
            pl.BlockSpec((1, d_out), lambda: (0, 0)),
        ],
        out_specs=pl.BlockSpec((n, d_out), lambda: (0, 0)),
        out_shape=jax.ShapeDtypeStruct((n, d_out), jnp.float32),
        scratch_shapes=[
            pltpu.VMEM((n, d_out), jnp.float32),
            pltpu.VMEM((nbuf, bs, n), jnp.float32),
            pltpu.SemaphoreType.DMA((nbuf,)),
        ],
    )(x, W, adj, b.reshape(1, d_out))

    return out[None]
